# baseline (speedup 1.0000x reference)
import functools

import jax
import jax.numpy as jnp
from jax import lax
from jax.experimental import pallas as pl
from jax.experimental.pallas import tpu as pltpu
from jax.experimental.pallas import tpu_sc as plsc

VOCAB = 100000
DIM = 128
SEQ = 50
SEQ_PAD = 56
NC = 2
NS = 16
NW = NC * NS
BPW = 4096 // NW
NBUF = 8
NGROUP = BPW // NBUF


def _body(x_hbm, tbl_hbm, out_hbm, idx_v, rows_v, gsem, osem):
    wid = lax.axis_index("s") * NC + lax.axis_index("c")
    pltpu.sync_copy(x_hbm.at[wid], idx_v)

    def start_gather(b, buf):
        pltpu.async_copy(
            tbl_hbm.at[idx_v.at[b, pl.ds(0, SEQ_PAD)]], rows_v.at[buf],
            gsem.at[buf])

    def wait_gather(buf):
        pltpu.make_async_copy(
            tbl_hbm.at[idx_v.at[0, pl.ds(0, SEQ_PAD)]], rows_v.at[buf],
            gsem.at[buf]).wait()

    def start_out(b, buf):
        pltpu.async_copy(
            rows_v.at[buf, pl.ds(0, SEQ)], out_hbm.at[wid * BPW + b],
            osem.at[buf])

    def wait_out(buf):
        pltpu.make_async_copy(
            rows_v.at[buf, pl.ds(0, SEQ)], out_hbm.at[0], osem.at[buf]).wait()

    for buf in range(NBUF):
        start_gather(buf, buf)

    def group(g, carry):
        for buf in range(NBUF):
            wait_gather(buf)
            start_out(g * NBUF + buf, buf)
        for buf in range(NBUF):
            wait_out(buf)

            @pl.when(g + 1 < NGROUP)
            def _():
                start_gather((g + 1) * NBUF + buf, buf)

        return carry

    lax.fori_loop(0, NGROUP, group, 0)


@jax.jit
def _run(x_pad, emb_weight):
    mesh = plsc.VectorSubcoreMesh(core_axis_name="c", subcore_axis_name="s")
    k = pl.kernel(
        _body,
        out_type=jax.ShapeDtypeStruct((4096, SEQ, DIM), jnp.float32),
        mesh=mesh,
        scratch_types=[
            pltpu.VMEM((BPW, 128), jnp.int32),
            pltpu.VMEM((NBUF, SEQ_PAD, DIM), jnp.float32),
            pltpu.SemaphoreType.DMA((NBUF,)),
            pltpu.SemaphoreType.DMA((NBUF,)),
        ],
        compiler_params=pltpu.CompilerParams(use_tc_tiling_on_sc=True),
    )
    return k(x_pad, emb_weight)


def kernel(x, emb_weight):
    b, s = x.shape
    x_pad = jnp.pad(x.astype(jnp.int32), ((0, 0), (0, 128 - s)))
    x_pad = x_pad.reshape(NW, BPW, 128)
    return _run(x_pad, emb_weight)
